# baseline (speedup 1.0000x reference)
import functools

import jax
import jax.numpy as jnp
from jax import lax
from jax.experimental import pallas as pl
from jax.experimental.pallas import tpu as pltpu
from jax.experimental.pallas import tpu_sc as plsc

_N = 100000
_WIN = 128
_TAIL0 = (_N // _WIN) * _WIN
_TAILW = _N - _TAIL0
_C0MAX = _TAIL0 - _WIN

_GATHER_DNUMS = lax.GatherDimensionNumbers(
    offset_dims=(), collapsed_slice_dims=(0,), start_index_map=(0,))


def _shuf(x, idx):
    return lax.gather(x, idx[:, None], _GATHER_DNUMS, (1,),
                      mode=lax.GatherScatterMode.PROMISE_IN_BOUNDS)


def _rsqrt(x):
    i = lax.bitcast_convert_type(x, jnp.int32)
    i = jnp.int32(0x5F3759DF) - (i >> 1)
    y = lax.bitcast_convert_type(i, jnp.float32)
    for _ in range(3):
        y = y * (1.5 - 0.5 * x * y * y)
    return y


_MESH = plsc.VectorSubcoreMesh(core_axis_name="c", subcore_axis_name="s", num_cores=1, num_subcores=1)


@functools.partial(
    pl.kernel,
    out_type=jax.ShapeDtypeStruct((16,), jnp.float32),
    mesh=_MESH,
    scratch_types=[
        pltpu.VMEM((16,), jnp.int32),
        pltpu.VMEM((6, _WIN), jnp.float32),
        pltpu.VMEM((6, _TAILW), jnp.float32),
        pltpu.VMEM((3, _WIN), jnp.float32),
        pltpu.VMEM((3, _TAILW), jnp.float32),
        pltpu.VMEM((16,), jnp.float32),
        pltpu.SemaphoreType.DMA,
    ],
)
def _pose_kernel(r6_hbm, t3_hbm, t_hbm, out_hbm,
                 tv, r6a_v, r6b_v, t3a_v, t3b_v, out_v, sem):
    cid = lax.axis_index("c")
    sid = lax.axis_index("s")

    @pl.when(jnp.logical_and(cid == 0, sid == 0))
    def _():
        pltpu.sync_copy(t_hbm, tv)
        vt = tv[...]
        lane = lax.iota(jnp.int32, 16)

        c0 = jnp.minimum((vt >> 7) << 7, _C0MAX)
        c0s = pl.multiple_of(c0[0], _WIN)
        cp_a = pltpu.async_copy(r6_hbm.at[:, pl.ds(c0s, _WIN)], r6a_v, sem)
        cp_b = pltpu.async_copy(r6_hbm.at[:, pl.ds(_TAIL0, _TAILW)], r6b_v, sem)
        cp_c = pltpu.async_copy(t3_hbm.at[:, pl.ds(c0s, _WIN)], t3a_v, sem)
        cp_d = pltpu.async_copy(t3_hbm.at[:, pl.ds(_TAIL0, _TAILW)], t3b_v, sem)
        cp_a.wait()
        cp_b.wait()
        cp_c.wait()
        cp_d.wait()

        tail = vt >= _TAIL0
        tailf = jnp.where(tail, 1.0, 0.0).astype(jnp.float32)
        mainf = 1.0 - tailf
        ca = vt - c0
        cb = vt - _TAIL0
        csel = jnp.where(tail, cb, ca)
        chunk = csel >> 4
        off = csel & 15
        wk = [jnp.where(chunk == k, 1.0, 0.0).astype(jnp.float32)
              for k in range(_WIN // 16)]

        def extract(buf_a, buf_b, j):
            val = buf_b[j, 0:16] * (tailf * wk[0])
            val = val + buf_b[j, 16:32] * (tailf * wk[1])
            for k in range(_WIN // 16):
                val = val + buf_a[j, 16 * k:16 * (k + 1)] * (mainf * wk[k])
            return _shuf(val, off)

        row6 = extract(r6a_v, r6b_v, 0)
        for j in range(1, 6):
            row6 = jnp.where(lane == j, extract(r6a_v, r6b_v, j), row6)
        row3 = extract(t3a_v, t3b_v, 0)
        for j in range(1, 3):
            row3 = jnp.where(lane == j, extract(t3a_v, t3b_v, j), row3)

        grp = (jnp.where(lane >= 3, 1, 0) + jnp.where(lane >= 6, 1, 0)
               + jnp.where(lane >= 9, 1, 0) + jnp.where(lane >= 12, 1, 0)
               + jnp.where(lane >= 15, 1, 0))
        g3 = grp * 3
        e = lane - g3
        v_ga = _shuf(row6, e + jnp.where(grp >= 2, 3, 0))
        v_gb = _shuf(row6, e + jnp.where(grp >= 1, 3, 0))
        prod = v_ga * v_gb
        one = jnp.int32(1)
        e1 = e + jnp.where(e >= 2, -2, one)
        e2 = e1 + jnp.where(e1 >= 2, -2, one)
        g1 = jnp.minimum(g3 + e1, 15)
        g2 = jnp.minimum(g3 + e2, 15)
        dots = prod + _shuf(prod, g1) + _shuf(prod, g2)
        zero = lane * 0
        xx = _shuf(dots, zero)
        xy = _shuf(dots, zero + 3)
        yy = _shuf(dots, zero + 6)

        s = xy / xx
        cc = yy - s * xy
        r1 = _rsqrt(xx)
        rc = _rsqrt(cc)

        pat_x = jnp.minimum(lane & 3, 5)
        v_x = _shuf(row6, pat_x)
        v_y = _shuf(row6, jnp.minimum(pat_x + 3, 5))
        m_b3 = jnp.logical_and(lane >= 8, lane < 11)
        q = lane - 8
        i1 = jnp.where(m_b3, lax.rem(q + 1, 3), 0)
        i2 = jnp.where(m_b3, lax.rem(q + 2, 3), 0)
        v_p = _shuf(row6, i1)
        v_q = _shuf(row6, i2 + 3)
        v_r = _shuf(row6, i2)
        v_s = _shuf(row6, i1 + 3)
        vb3 = (r1 * rc) * (v_p * v_q - v_r * v_s)
        m_t = jnp.logical_or(jnp.logical_or(lane == 3, lane == 7), lane == 11)
        v_t = _shuf(row3, jnp.where(m_t, (lane - 3) >> 2, 0))

        m_b1 = lane < 3
        m_b2 = jnp.logical_and(lane >= 4, lane < 7)
        tailc = jnp.where(lane == 15, 1.0, 0.0).astype(jnp.float32)
        out = jnp.where(
            m_b1,
            r1 * v_x,
            jnp.where(
                m_b2,
                rc * (v_y - s * v_x),
                jnp.where(m_b3, vb3, jnp.where(m_t, v_t, tailc)),
            ),
        )
        out_v[...] = out
        pltpu.sync_copy(out_v, out_hbm)


def kernel(R_6d, T, t):
    r6t = R_6d.T
    t3t = T.T
    tvec = jnp.full((16,), t, dtype=jnp.int32)
    flat = _pose_kernel(r6t, t3t, tvec)
    return flat.reshape(4, 4)


# speedup vs baseline: 1.0067x; 1.0067x over previous
import functools

import jax
import jax.numpy as jnp
from jax import lax
from jax.experimental import pallas as pl
from jax.experimental.pallas import tpu as pltpu
from jax.experimental.pallas import tpu_sc as plsc

_N = 100000
_WIN = 128
_TAIL0 = (_N // _WIN) * _WIN
_TAILW = _N - _TAIL0
_C0MAX = _TAIL0 - _WIN

_GATHER_DNUMS = lax.GatherDimensionNumbers(
    offset_dims=(), collapsed_slice_dims=(0,), start_index_map=(0,))


def _shuf(x, idx):
    return lax.gather(x, idx[:, None], _GATHER_DNUMS, (1,),
                      mode=lax.GatherScatterMode.PROMISE_IN_BOUNDS)


def _rsqrt(x):
    i = lax.bitcast_convert_type(x, jnp.int32)
    i = jnp.int32(0x5F3759DF) - (i >> 1)
    y = lax.bitcast_convert_type(i, jnp.float32)
    for _ in range(3):
        y = y * (1.5 - 0.5 * x * y * y)
    return y


_MESH = plsc.VectorSubcoreMesh(core_axis_name="c", subcore_axis_name="s", num_cores=1, num_subcores=1)


@functools.partial(
    pl.kernel,
    out_type=jax.ShapeDtypeStruct((16,), jnp.float32),
    mesh=_MESH,
    scratch_types=[
        pltpu.VMEM((16,), jnp.int32),
        pltpu.VMEM((6, _WIN), jnp.float32),
        pltpu.VMEM((6, _TAILW), jnp.float32),
        pltpu.VMEM((3, _WIN), jnp.float32),
        pltpu.VMEM((3, _TAILW), jnp.float32),
        pltpu.VMEM((16,), jnp.float32),
        pltpu.SemaphoreType.DMA,
    ],
)
def _pose_kernel(r6_hbm, t3_hbm, t_hbm, out_hbm,
                 tv, r6a_v, r6b_v, t3a_v, t3b_v, out_v, sem):
    cid = lax.axis_index("c")
    sid = lax.axis_index("s")

    @pl.when(jnp.logical_and(cid == 0, sid == 0))
    def _():
        pltpu.sync_copy(t_hbm, tv)
        vt = tv[...]
        lane = lax.iota(jnp.int32, 16)

        c0 = jnp.minimum((vt >> 7) << 7, _C0MAX)
        c0s = pl.multiple_of(c0[0], _WIN)
        is_tail = vt[0] >= _TAIL0
        cp_a = pltpu.async_copy(r6_hbm.at[:, pl.ds(c0s, _WIN)], r6a_v, sem)
        cp_c = pltpu.async_copy(t3_hbm.at[:, pl.ds(c0s, _WIN)], t3a_v, sem)

        @pl.when(is_tail)
        def _fetch_tail():
            cp_b = pltpu.async_copy(
                r6_hbm.at[:, pl.ds(_TAIL0, _TAILW)], r6b_v, sem)
            cp_d = pltpu.async_copy(
                t3_hbm.at[:, pl.ds(_TAIL0, _TAILW)], t3b_v, sem)
            cp_b.wait()
            cp_d.wait()

        cp_a.wait()
        cp_c.wait()

        tail = vt >= _TAIL0
        ca = vt - c0
        cb = vt - _TAIL0
        csel = jnp.where(tail, cb, ca)
        chunk = csel >> 4
        off = csel & 15
        wk = [jnp.where(chunk == k, 1.0, 0.0).astype(jnp.float32)
              for k in range(_WIN // 16)]

        def extract(buf_a, buf_b, j):
            val_tail = buf_b[j, 0:16] * wk[0] + buf_b[j, 16:32] * wk[1]
            val_main = buf_a[j, 0:16] * wk[0]
            for k in range(1, _WIN // 16):
                val_main = val_main + buf_a[j, 16 * k:16 * (k + 1)] * wk[k]
            return _shuf(jnp.where(tail, val_tail, val_main), off)

        row6 = extract(r6a_v, r6b_v, 0)
        for j in range(1, 6):
            row6 = jnp.where(lane == j, extract(r6a_v, r6b_v, j), row6)
        row3 = extract(t3a_v, t3b_v, 0)
        for j in range(1, 3):
            row3 = jnp.where(lane == j, extract(t3a_v, t3b_v, j), row3)

        grp = (jnp.where(lane >= 3, 1, 0) + jnp.where(lane >= 6, 1, 0)
               + jnp.where(lane >= 9, 1, 0) + jnp.where(lane >= 12, 1, 0)
               + jnp.where(lane >= 15, 1, 0))
        g3 = grp * 3
        e = lane - g3
        v_ga = _shuf(row6, e + jnp.where(grp >= 2, 3, 0))
        v_gb = _shuf(row6, e + jnp.where(grp >= 1, 3, 0))
        prod = v_ga * v_gb
        one = jnp.int32(1)
        e1 = e + jnp.where(e >= 2, -2, one)
        e2 = e1 + jnp.where(e1 >= 2, -2, one)
        g1 = jnp.minimum(g3 + e1, 15)
        g2 = jnp.minimum(g3 + e2, 15)
        dots = prod + _shuf(prod, g1) + _shuf(prod, g2)
        zero = lane * 0
        xx = _shuf(dots, zero)
        xy = _shuf(dots, zero + 3)
        yy = _shuf(dots, zero + 6)

        s = xy / xx
        cc = yy - s * xy
        r1 = _rsqrt(xx)
        rc = _rsqrt(cc)

        pat_x = jnp.minimum(lane & 3, 5)
        v_x = _shuf(row6, pat_x)
        v_y = _shuf(row6, jnp.minimum(pat_x + 3, 5))
        m_b3 = jnp.logical_and(lane >= 8, lane < 11)
        q = lane - 8
        i1 = jnp.where(m_b3, lax.rem(q + 1, 3), 0)
        i2 = jnp.where(m_b3, lax.rem(q + 2, 3), 0)
        v_p = _shuf(row6, i1)
        v_q = _shuf(row6, i2 + 3)
        v_r = _shuf(row6, i2)
        v_s = _shuf(row6, i1 + 3)
        vb3 = (r1 * rc) * (v_p * v_q - v_r * v_s)
        m_t = jnp.logical_or(jnp.logical_or(lane == 3, lane == 7), lane == 11)
        v_t = _shuf(row3, jnp.where(m_t, (lane - 3) >> 2, 0))

        m_b1 = lane < 3
        m_b2 = jnp.logical_and(lane >= 4, lane < 7)
        tailc = jnp.where(lane == 15, 1.0, 0.0).astype(jnp.float32)
        out = jnp.where(
            m_b1,
            r1 * v_x,
            jnp.where(
                m_b2,
                rc * (v_y - s * v_x),
                jnp.where(m_b3, vb3, jnp.where(m_t, v_t, tailc)),
            ),
        )
        out_v[...] = out
        pltpu.sync_copy(out_v, out_hbm)


def kernel(R_6d, T, t):
    r6t = R_6d.T
    t3t = T.T
    tvec = jnp.full((16,), t, dtype=jnp.int32)
    flat = _pose_kernel(r6t, t3t, tvec)
    return flat.reshape(4, 4)
